# Initial kernel scaffold; baseline (speedup 1.0000x reference)
import jax
import jax.numpy as jnp
from jax.experimental import pallas as pl


def kernel(x_in, edge_index, edge_attr, params):
    raise NotImplementedError("write your pallas kernel here")


# speedup vs baseline: 2.7180x; 2.7180x over previous
import functools

import jax
import jax.numpy as jnp
from jax import lax
from jax.experimental import pallas as pl
from jax.experimental.pallas import tpu as pltpu
from jax.experimental.pallas import tpu_sc as plsc

f32 = jnp.float32
i32 = jnp.int32

N = 10000
E = 320000
IND = 128
OUTD = 64
EDGED = 16
H1 = 32

NC = 2
NS = 16
NW = NC * NS
EPW = E // NW
CH = 80
NCHUNK = EPW // CH
NPAD = 10240
RPT = NPAD // NS

BE = 4000
GE = E // BE
BN = 2000
GN = N // BN


def _ln_k(x, g, b, eps=1e-6):
    m = jnp.mean(x, axis=-1, keepdims=True)
    v = jnp.mean((x - m) ** 2, axis=-1, keepdims=True)
    return (x - m) * lax.rsqrt(v + eps) * g + b


def _full(shape):
    return pl.BlockSpec(shape, lambda i: tuple(0 for _ in shape))


def _edge_enc_body(ea, ge, be, W1t, b1, W2t, b2, W3t, b3, Wc1t, bc1, Wc2t,
                   bc2, out):
    a = ea[...]
    h = _ln_k(a, ge[...], be[...])
    h = jnp.maximum(h @ W1t[...] + b1[...], 0.0)
    h = jnp.maximum(h @ W2t[...] + b2[...], 0.0)
    enc = h @ W3t[...] + b3[...]
    c = jnp.maximum(a @ Wc1t[...] + bc1[...], 0.0)
    w = jax.nn.sigmoid(c @ Wc2t[...] + bc2[...])
    out[...] = enc * w


def _edge_enc(ea, *ws):
    specs = [pl.BlockSpec((BE, EDGED), lambda i: (i, 0))]
    specs += [_full(w.shape) for w in ws]
    return pl.pallas_call(
        _edge_enc_body,
        grid=(GE,),
        in_specs=specs,
        out_specs=pl.BlockSpec((BE, OUTD), lambda i: (i, 0)),
        out_shape=jax.ShapeDtypeStruct((E, OUTD), f32),
    )(ea, *ws)


def _node_body(x_ref, dummy, g0, b0, Wst, bs, Wgt, bg,
               xn_ref, gate_ref, gskip_ref):
    x = x_ref[...]
    bad = x[:, 0:1] == -999.0
    x = jnp.where(bad, dummy[...], x)
    xn = _ln_k(x, g0[...], b0[...])
    xn_ref[...] = xn
    skip = xn @ Wst[...] + bs[...]
    gate = jax.nn.sigmoid(skip @ Wgt[...] + bg[...])
    gate_ref[...] = gate
    gskip_ref[...] = gate * skip


def _node(x, *ws):
    specs = [pl.BlockSpec((BN, IND), lambda i: (i, 0))]
    specs += [_full(w.shape) for w in ws]
    return pl.pallas_call(
        _node_body,
        grid=(GN,),
        in_specs=specs,
        out_specs=[
            pl.BlockSpec((BN, IND), lambda i: (i, 0)),
            pl.BlockSpec((BN, OUTD), lambda i: (i, 0)),
            pl.BlockSpec((BN, OUTD), lambda i: (i, 0)),
        ],
        out_shape=[
            jax.ShapeDtypeStruct((N, IND), f32),
            jax.ShapeDtypeStruct((N, OUTD), f32),
            jax.ShapeDtypeStruct((N, OUTD), f32),
        ],
    )(x, *ws)


def _mlp1_body(gd, gs, ee, Adt, Ast, Aet, b1a, W1bt, b1b, W1ct, b1c, out):
    m = jnp.maximum(gd[...] @ Adt[...] + gs[...] @ Ast[...]
                    + ee[...] @ Aet[...] + b1a[...], 0.0)
    m = jnp.maximum(m @ W1bt[...] + b1b[...], 0.0)
    m = m @ W1ct[...] + b1c[...]
    colid = lax.broadcasted_iota(i32, (BE, IND - H1), 1)
    aug = jnp.where(colid == 0, 1.0, 0.0).astype(f32)
    out[...] = jnp.concatenate([m, aug], axis=1)


def _mlp1(gd, gs, ee, *ws):
    specs = [
        pl.BlockSpec((BE, IND), lambda i: (i, 0)),
        pl.BlockSpec((BE, IND), lambda i: (i, 0)),
        pl.BlockSpec((BE, OUTD), lambda i: (i, 0)),
    ]
    specs += [_full(w.shape) for w in ws]
    return pl.pallas_call(
        _mlp1_body,
        grid=(GE,),
        in_specs=specs,
        out_specs=pl.BlockSpec((BE, IND), lambda i: (i, 0)),
        out_shape=jax.ShapeDtypeStruct((E, IND), f32),
    )(gd, gs, ee, *ws)


def _x1_body(pa, pb, g1, b1, x1_ref, invd_ref):
    s = pa[0] + pb[0]
    cnt = s[:, H1:H1 + 1]
    invd = 1.0 / jnp.maximum(cnt, 1.0)
    z = _ln_k(s[:, :H1] * invd, g1[...], b1[...])
    z = jnp.where(z >= 0.0, z, 0.01 * z)
    x1_ref[...] = jnp.concatenate(
        [z, jnp.zeros((BN, IND - H1), f32)], axis=1)
    invd_ref[...] = invd


def _x1(s1, g1, b1):
    return pl.pallas_call(
        _x1_body,
        grid=(GN,),
        in_specs=[
            pl.BlockSpec((1, BN, IND), lambda i: (0, i, 0)),
            pl.BlockSpec((1, BN, IND), lambda i: (1, i, 0)),
            _full(g1.shape),
            _full(b1.shape),
        ],
        out_specs=[
            pl.BlockSpec((BN, IND), lambda i: (i, 0)),
            pl.BlockSpec((BN, 1), lambda i: (i, 0)),
        ],
        out_shape=[
            jax.ShapeDtypeStruct((N, IND), f32),
            jax.ShapeDtypeStruct((N, 1), f32),
        ],
    )(s1, s1, g1, b1)


def _mlp2_body(xd, xs, ee, Bdt, Bst, Bet, b2a, W2bt, b2b, W2ct, b2c, out):
    e = ee[...]
    m = jnp.maximum(xd[...] @ Bdt[...] + xs[...] @ Bst[...]
                    + e @ Bet[...] + b2a[...], 0.0)
    m = jnp.maximum(m @ W2bt[...] + b2b[...], 0.0)
    m = m @ W2ct[...] + b2c[...]
    out[...] = jnp.concatenate([m, e], axis=1)


def _mlp2(xd, xs, ee, *ws):
    specs = [
        pl.BlockSpec((BE, IND), lambda i: (i, 0)),
        pl.BlockSpec((BE, IND), lambda i: (i, 0)),
        pl.BlockSpec((BE, OUTD), lambda i: (i, 0)),
    ]
    specs += [_full(w.shape) for w in ws]
    return pl.pallas_call(
        _mlp2_body,
        grid=(GE,),
        in_specs=specs,
        out_specs=pl.BlockSpec((BE, IND), lambda i: (i, 0)),
        out_shape=jax.ShapeDtypeStruct((E, IND), f32),
    )(xd, xs, ee, *ws)


def _final_body(p2a, p2b, invd, gate, gskip, g2, b2,
                Wp1t, bp1, Wp2t, bp2, Wp3t, bp3, xfc_ref, probs_ref):
    inv = invd[...]
    s = (p2a[0] + p2b[0]) * inv
    x2 = _ln_k(s[:, :OUTD], g2[...], b2[...])
    x2 = jnp.maximum(x2, 0.0)
    efm = s[:, OUTD:]
    g = gate[...]
    xf = gskip[...] + (1.0 - g) * x2
    xfc = jnp.concatenate([xf, efm], axis=1)
    xfc_ref[...] = xfc
    h = xfc @ Wp1t[...] + bp1[...]
    h = jnp.where(h > 0.0, h, jnp.exp(h) - 1.0)
    h = h @ Wp2t[...] + bp2[...]
    h = jnp.where(h > 0.0, h, jnp.exp(h) - 1.0)
    probs_ref[...] = h @ Wp3t[...] + bp3[...]


def _final(s2, invd, gate, gskip, *ws):
    specs = [
        pl.BlockSpec((1, BN, IND), lambda i: (0, i, 0)),
        pl.BlockSpec((1, BN, IND), lambda i: (1, i, 0)),
        pl.BlockSpec((BN, 1), lambda i: (i, 0)),
        pl.BlockSpec((BN, OUTD), lambda i: (i, 0)),
        pl.BlockSpec((BN, OUTD), lambda i: (i, 0)),
    ]
    specs += [_full(w.shape) for w in ws]
    return pl.pallas_call(
        _final_body,
        grid=(GN,),
        in_specs=specs,
        out_specs=[
            pl.BlockSpec((BN, 2 * OUTD), lambda i: (i, 0)),
            pl.BlockSpec((BN, 1), lambda i: (i, 0)),
        ],
        out_shape=[
            jax.ShapeDtypeStruct((N, 2 * OUTD), f32),
            jax.ShapeDtypeStruct((N, 1), f32),
        ],
    )(s2, s2, invd, gate, gskip, *ws)


_sc_mesh = plsc.VectorSubcoreMesh(
    core_axis_name="c", subcore_axis_name="s", num_cores=NC, num_subcores=NS)


@functools.partial(
    pl.kernel,
    out_type=(jax.ShapeDtypeStruct((E, IND), f32),
              jax.ShapeDtypeStruct((E, IND), f32)),
    mesh=_sc_mesh,
    scratch_types=[
        pltpu.VMEM((CH,), i32), pltpu.VMEM((CH,), i32),
        pltpu.VMEM((CH, IND), f32), pltpu.VMEM((CH, IND), f32),
        pltpu.SemaphoreType.DMA, pltpu.SemaphoreType.DMA,
    ],
)
def _sc_gather_pair(ta, tb, ia, ib, oa, ob,
                    idxa, idxb, rowsa, rowsb, sema, semb):
    c = lax.axis_index("c")
    s = lax.axis_index("s")
    wid = c * NS + s

    def body(j, carry):
        base = wid * EPW + j * CH
        pltpu.sync_copy(ia.at[pl.ds(base, CH)], idxa)
        pltpu.sync_copy(ib.at[pl.ds(base, CH)], idxb)
        cpa = pltpu.async_copy(ta.at[idxa], rowsa, sema)
        cpb = pltpu.async_copy(tb.at[idxb], rowsb, semb)
        cpa.wait()
        cpb.wait()
        pltpu.sync_copy(rowsa, oa.at[pl.ds(base, CH)])
        pltpu.sync_copy(rowsb, ob.at[pl.ds(base, CH)])
        return carry

    lax.fori_loop(0, NCHUNK, body, 0)


@functools.partial(
    pl.kernel,
    out_type=jax.ShapeDtypeStruct((NC, NPAD, IND), f32),
    mesh=_sc_mesh,
    scratch_types=[
        pltpu.VMEM((CH,), i32), pltpu.VMEM((CH, IND), f32),
        pltpu.VMEM_SHARED((NPAD, IND), f32),
    ],
)
def _sc_scatter128(vals, dsti, zer, out, idx_v, rows_v, acc):
    c = lax.axis_index("c")
    s = lax.axis_index("s")
    r0 = s * RPT
    pltpu.sync_copy(zer.at[pl.ds(r0, RPT)], acc.at[pl.ds(r0, RPT)])
    plsc.subcore_barrier()
    wid = c * NS + s

    def body(j, carry):
        base = wid * EPW + j * CH
        pltpu.sync_copy(dsti.at[pl.ds(base, CH)], idx_v)
        pltpu.sync_copy(vals.at[pl.ds(base, CH)], rows_v)
        pltpu.sync_copy(rows_v, acc.at[idx_v], add=True)
        return carry

    lax.fori_loop(0, NCHUNK, body, 0)
    plsc.subcore_barrier()
    pltpu.sync_copy(acc.at[pl.ds(r0, RPT)], out.at[c, pl.ds(r0, RPT)])


def kernel(x_in, edge_index, edge_attr, params):
    p = params
    src = edge_index[0, 0]
    dst = edge_index[0, 1]
    x = x_in[0]
    ea = edge_attr[0]

    def r(v):
        return v.reshape(1, -1)

    Adt = p['Wm1a'][:, :IND].T
    Ast = p['Wm1a'][:, IND:2 * IND].T
    Aet = p['Wm1a'][:, 2 * IND:].T
    Bdt = jnp.zeros((IND, OUTD), f32).at[:H1].set(p['Wm2a'][:, :H1].T)
    Bst = jnp.zeros((IND, OUTD), f32).at[:H1].set(p['Wm2a'][:, H1:2 * H1].T)
    Bet = p['Wm2a'][:, 2 * H1:].T
    zer = jnp.zeros((NPAD, IND), f32)

    e_enc = _edge_enc(ea, r(p['ge']), r(p['be']),
                      p['We1'].T, r(p['be1']), p['We2'].T, r(p['be2']),
                      p['We3'].T, r(p['be3']),
                      p['Wc1'].T, r(p['bc1']), p['Wc2'].T, r(p['bc2']))
    xn, gate, gskip = _node(x, r(p['dummy']), r(p['g0']), r(p['b0']),
                            p['Wskip'].T, r(p['bskip']),
                            p['Wg'].T, r(p['bg']))
    gd, gs = _sc_gather_pair(xn, xn, dst, src)
    m1 = _mlp1(gd, gs, e_enc, Adt, Ast, Aet, r(p['bm1a']),
               p['Wm1b'].T, r(p['bm1b']), p['Wm1c'].T, r(p['bm1c']))
    s1 = _sc_scatter128(m1, dst, zer)
    x1, invd = _x1(s1, r(p['g1']), r(p['b1']))
    xd1, xs1 = _sc_gather_pair(x1, x1, dst, src)
    m2 = _mlp2(xd1, xs1, e_enc, Bdt, Bst, Bet, r(p['bm2a']),
               p['Wm2b'].T, r(p['bm2b']), p['Wm2c'].T, r(p['bm2c']))
    s2 = _sc_scatter128(m2, dst, zer)
    xfc, probs = _final(s2, invd, gate, gskip,
                        r(p['g2']), r(p['b2']),
                        p['Wp1'].T, r(p['bp1']), p['Wp2'].T, r(p['bp2']),
                        p['Wp3'].T, r(p['bp3']))
    return (xfc[None], probs[None], jnp.zeros((1,), f32))


# speedup vs baseline: 2.7903x; 1.0266x over previous
import functools

import jax
import jax.numpy as jnp
from jax import lax
from jax.experimental import pallas as pl
from jax.experimental.pallas import tpu as pltpu
from jax.experimental.pallas import tpu_sc as plsc

f32 = jnp.float32
i32 = jnp.int32

N = 10000
E = 320000
IND = 128
OUTD = 64
EDGED = 16
H1 = 32

NC = 2
NS = 16
NW = NC * NS
EPW = E // NW
CH = 80
NCHUNK = EPW // CH
NPAD = 10240
RPT = NPAD // NS

BE = 4000
GE = E // BE
BN = 2000
GN = N // BN


def _ln_k(x, g, b, eps=1e-6):
    m = jnp.mean(x, axis=-1, keepdims=True)
    v = jnp.mean((x - m) ** 2, axis=-1, keepdims=True)
    return (x - m) * lax.rsqrt(v + eps) * g + b


def _full(shape):
    return pl.BlockSpec(shape, lambda i: tuple(0 for _ in shape))


def _edge_enc_body(ea, ge, be, W1t, b1, W2t, b2, W3t, b3, Wc1t, bc1, Wc2t,
                   bc2, out):
    a = ea[...]
    h = _ln_k(a, ge[...], be[...])
    h = jnp.maximum(h @ W1t[...] + b1[...], 0.0)
    h = jnp.maximum(h @ W2t[...] + b2[...], 0.0)
    enc = h @ W3t[...] + b3[...]
    c = jnp.maximum(a @ Wc1t[...] + bc1[...], 0.0)
    w = jax.nn.sigmoid(c @ Wc2t[...] + bc2[...])
    out[...] = enc * w


def _edge_enc(ea, *ws):
    specs = [pl.BlockSpec((BE, EDGED), lambda i: (i, 0))]
    specs += [_full(w.shape) for w in ws]
    return pl.pallas_call(
        _edge_enc_body,
        grid=(GE,),
        in_specs=specs,
        out_specs=pl.BlockSpec((BE, OUTD), lambda i: (i, 0)),
        out_shape=jax.ShapeDtypeStruct((E, OUTD), f32),
    )(ea, *ws)


def _node_body(x_ref, dummy, g0, b0, Adt, Ast, Wst, bs, Wgt, bg,
               pd_ref, ps_ref, gate_ref, gskip_ref):
    x = x_ref[...]
    bad = x[:, 0:1] == -999.0
    x = jnp.where(bad, dummy[...], x)
    xn = _ln_k(x, g0[...], b0[...])
    zpad = jnp.zeros((BN, IND - H1), f32)
    pd_ref[...] = jnp.concatenate([xn @ Adt[...], zpad], axis=1)
    ps_ref[...] = jnp.concatenate([xn @ Ast[...], zpad], axis=1)
    skip = xn @ Wst[...] + bs[...]
    gate = jax.nn.sigmoid(skip @ Wgt[...] + bg[...])
    gate_ref[...] = gate
    gskip_ref[...] = gate * skip


def _node(x, *ws):
    specs = [pl.BlockSpec((BN, IND), lambda i: (i, 0))]
    specs += [_full(w.shape) for w in ws]
    return pl.pallas_call(
        _node_body,
        grid=(GN,),
        in_specs=specs,
        out_specs=[
            pl.BlockSpec((BN, IND), lambda i: (i, 0)),
            pl.BlockSpec((BN, IND), lambda i: (i, 0)),
            pl.BlockSpec((BN, OUTD), lambda i: (i, 0)),
            pl.BlockSpec((BN, OUTD), lambda i: (i, 0)),
        ],
        out_shape=[
            jax.ShapeDtypeStruct((N, IND), f32),
            jax.ShapeDtypeStruct((N, IND), f32),
            jax.ShapeDtypeStruct((N, OUTD), f32),
            jax.ShapeDtypeStruct((N, OUTD), f32),
        ],
    )(x, *ws)


def _mlp1_body(pre, ee, Aet, b1a, W1bt, b1b, W1ct, b1c, out):
    m = jnp.maximum(pre[...][:, :H1] + ee[...] @ Aet[...] + b1a[...], 0.0)
    m = jnp.maximum(m @ W1bt[...] + b1b[...], 0.0)
    m = m @ W1ct[...] + b1c[...]
    colid = lax.broadcasted_iota(i32, (BE, IND - H1), 1)
    aug = jnp.where(colid == 0, 1.0, 0.0).astype(f32)
    out[...] = jnp.concatenate([m, aug], axis=1)


def _mlp1(pre, ee, *ws):
    specs = [
        pl.BlockSpec((BE, IND), lambda i: (i, 0)),
        pl.BlockSpec((BE, OUTD), lambda i: (i, 0)),
    ]
    specs += [_full(w.shape) for w in ws]
    return pl.pallas_call(
        _mlp1_body,
        grid=(GE,),
        in_specs=specs,
        out_specs=pl.BlockSpec((BE, IND), lambda i: (i, 0)),
        out_shape=jax.ShapeDtypeStruct((E, IND), f32),
    )(pre, ee, *ws)


def _x1_body(pa, pb, g1, b1, Bdt, Bst, qd_ref, qs_ref, invd_ref):
    s = pa[0] + pb[0]
    cnt = s[:, H1:H1 + 1]
    invd = 1.0 / jnp.maximum(cnt, 1.0)
    z = _ln_k(s[:, :H1] * invd, g1[...], b1[...])
    z = jnp.where(z >= 0.0, z, 0.01 * z)
    zpad = jnp.zeros((BN, IND - OUTD), f32)
    qd_ref[...] = jnp.concatenate([z @ Bdt[...], zpad], axis=1)
    qs_ref[...] = jnp.concatenate([z @ Bst[...], zpad], axis=1)
    invd_ref[...] = invd


def _x1(s1, g1, b1, Bdt, Bst):
    return pl.pallas_call(
        _x1_body,
        grid=(GN,),
        in_specs=[
            pl.BlockSpec((1, BN, IND), lambda i: (0, i, 0)),
            pl.BlockSpec((1, BN, IND), lambda i: (1, i, 0)),
            _full(g1.shape),
            _full(b1.shape),
            _full(Bdt.shape),
            _full(Bst.shape),
        ],
        out_specs=[
            pl.BlockSpec((BN, IND), lambda i: (i, 0)),
            pl.BlockSpec((BN, IND), lambda i: (i, 0)),
            pl.BlockSpec((BN, 1), lambda i: (i, 0)),
        ],
        out_shape=[
            jax.ShapeDtypeStruct((N, IND), f32),
            jax.ShapeDtypeStruct((N, IND), f32),
            jax.ShapeDtypeStruct((N, 1), f32),
        ],
    )(s1, s1, g1, b1, Bdt, Bst)


def _mlp2_body(pre, ee, Bet, b2a, W2bt, b2b, W2ct, b2c, out):
    e = ee[...]
    m = jnp.maximum(pre[...][:, :OUTD] + e @ Bet[...] + b2a[...], 0.0)
    m = jnp.maximum(m @ W2bt[...] + b2b[...], 0.0)
    m = m @ W2ct[...] + b2c[...]
    out[...] = jnp.concatenate([m, e], axis=1)


def _mlp2(pre, ee, *ws):
    specs = [
        pl.BlockSpec((BE, IND), lambda i: (i, 0)),
        pl.BlockSpec((BE, OUTD), lambda i: (i, 0)),
    ]
    specs += [_full(w.shape) for w in ws]
    return pl.pallas_call(
        _mlp2_body,
        grid=(GE,),
        in_specs=specs,
        out_specs=pl.BlockSpec((BE, IND), lambda i: (i, 0)),
        out_shape=jax.ShapeDtypeStruct((E, IND), f32),
    )(pre, ee, *ws)


def _final_body(p2a, p2b, invd, gate, gskip, g2, b2,
                Wp1t, bp1, Wp2t, bp2, Wp3t, bp3, xfc_ref, probs_ref):
    inv = invd[...]
    s = (p2a[0] + p2b[0]) * inv
    x2 = _ln_k(s[:, :OUTD], g2[...], b2[...])
    x2 = jnp.maximum(x2, 0.0)
    efm = s[:, OUTD:]
    g = gate[...]
    xf = gskip[...] + (1.0 - g) * x2
    xfc = jnp.concatenate([xf, efm], axis=1)
    xfc_ref[...] = xfc
    h = xfc @ Wp1t[...] + bp1[...]
    h = jnp.where(h > 0.0, h, jnp.exp(h) - 1.0)
    h = h @ Wp2t[...] + bp2[...]
    h = jnp.where(h > 0.0, h, jnp.exp(h) - 1.0)
    probs_ref[...] = h @ Wp3t[...] + bp3[...]


def _final(s2, invd, gate, gskip, *ws):
    specs = [
        pl.BlockSpec((1, BN, IND), lambda i: (0, i, 0)),
        pl.BlockSpec((1, BN, IND), lambda i: (1, i, 0)),
        pl.BlockSpec((BN, 1), lambda i: (i, 0)),
        pl.BlockSpec((BN, OUTD), lambda i: (i, 0)),
        pl.BlockSpec((BN, OUTD), lambda i: (i, 0)),
    ]
    specs += [_full(w.shape) for w in ws]
    return pl.pallas_call(
        _final_body,
        grid=(GN,),
        in_specs=specs,
        out_specs=[
            pl.BlockSpec((BN, 2 * OUTD), lambda i: (i, 0)),
            pl.BlockSpec((BN, 1), lambda i: (i, 0)),
        ],
        out_shape=[
            jax.ShapeDtypeStruct((N, 2 * OUTD), f32),
            jax.ShapeDtypeStruct((N, 1), f32),
        ],
    )(s2, s2, invd, gate, gskip, *ws)


_sc_mesh = plsc.VectorSubcoreMesh(
    core_axis_name="c", subcore_axis_name="s", num_cores=NC, num_subcores=NS)


@functools.partial(
    pl.kernel,
    out_type=jax.ShapeDtypeStruct((E, IND), f32),
    mesh=_sc_mesh,
    scratch_types=[
        pltpu.VMEM((CH,), i32), pltpu.VMEM((CH,), i32),
        pltpu.VMEM((CH, IND), f32),
        pltpu.SemaphoreType.DMA,
    ],
)
def _sc_gather_add(ta, tb, ia, ib, out, idxa, idxb, rows, sem):
    c = lax.axis_index("c")
    s = lax.axis_index("s")
    wid = c * NS + s

    def body(j, carry):
        base = wid * EPW + j * CH
        pltpu.sync_copy(ia.at[pl.ds(base, CH)], idxa)
        pltpu.sync_copy(ib.at[pl.ds(base, CH)], idxb)
        pltpu.async_copy(ta.at[idxa], rows, sem).wait()
        pltpu.async_copy(tb.at[idxb], rows, sem, add=True).wait()
        pltpu.sync_copy(rows, out.at[pl.ds(base, CH)])
        return carry

    lax.fori_loop(0, NCHUNK, body, 0)


@functools.partial(
    pl.kernel,
    out_type=jax.ShapeDtypeStruct((NC, NPAD, IND), f32),
    mesh=_sc_mesh,
    scratch_types=[
        pltpu.VMEM((CH,), i32), pltpu.VMEM((CH, IND), f32),
        pltpu.VMEM_SHARED((NPAD, IND), f32),
    ],
)
def _sc_scatter128(vals, dsti, zer, out, idx_v, rows_v, acc):
    c = lax.axis_index("c")
    s = lax.axis_index("s")
    r0 = s * RPT
    pltpu.sync_copy(zer.at[pl.ds(r0, RPT)], acc.at[pl.ds(r0, RPT)])
    plsc.subcore_barrier()
    wid = c * NS + s

    def body(j, carry):
        base = wid * EPW + j * CH
        pltpu.sync_copy(dsti.at[pl.ds(base, CH)], idx_v)
        pltpu.sync_copy(vals.at[pl.ds(base, CH)], rows_v)
        pltpu.sync_copy(rows_v, acc.at[idx_v], add=True)
        return carry

    lax.fori_loop(0, NCHUNK, body, 0)
    plsc.subcore_barrier()
    pltpu.sync_copy(acc.at[pl.ds(r0, RPT)], out.at[c, pl.ds(r0, RPT)])


def kernel(x_in, edge_index, edge_attr, params):
    p = params
    src = edge_index[0, 0]
    dst = edge_index[0, 1]
    x = x_in[0]
    ea = edge_attr[0]

    def r(v):
        return v.reshape(1, -1)

    Adt = p['Wm1a'][:, :IND].T
    Ast = p['Wm1a'][:, IND:2 * IND].T
    Aet = p['Wm1a'][:, 2 * IND:].T
    Bdt = p['Wm2a'][:, :H1].T
    Bst = p['Wm2a'][:, H1:2 * H1].T
    Bet = p['Wm2a'][:, 2 * H1:].T
    zer = jnp.zeros((NPAD, IND), f32)

    e_enc = _edge_enc(ea, r(p['ge']), r(p['be']),
                      p['We1'].T, r(p['be1']), p['We2'].T, r(p['be2']),
                      p['We3'].T, r(p['be3']),
                      p['Wc1'].T, r(p['bc1']), p['Wc2'].T, r(p['bc2']))
    pd, ps, gate, gskip = _node(x, r(p['dummy']), r(p['g0']), r(p['b0']),
                                Adt, Ast,
                                p['Wskip'].T, r(p['bskip']),
                                p['Wg'].T, r(p['bg']))
    m1pre = _sc_gather_add(pd, ps, dst, src)
    m1 = _mlp1(m1pre, e_enc, Aet, r(p['bm1a']),
               p['Wm1b'].T, r(p['bm1b']), p['Wm1c'].T, r(p['bm1c']))
    s1 = _sc_scatter128(m1, dst, zer)
    qd, qs, invd = _x1(s1, r(p['g1']), r(p['b1']), Bdt, Bst)
    m2pre = _sc_gather_add(qd, qs, dst, src)
    m2 = _mlp2(m2pre, e_enc, Bet, r(p['bm2a']),
               p['Wm2b'].T, r(p['bm2b']), p['Wm2c'].T, r(p['bm2c']))
    s2 = _sc_scatter128(m2, dst, zer)
    xfc, probs = _final(s2, invd, gate, gskip,
                        r(p['g2']), r(p['b2']),
                        p['Wp1'].T, r(p['bp1']), p['Wp2'].T, r(p['bp2']),
                        p['Wp3'].T, r(p['bp3']))
    return (xfc[None], probs[None], jnp.zeros((1,), f32))


# speedup vs baseline: 4.3575x; 1.5617x over previous
import functools

import jax
import jax.numpy as jnp
from jax import lax
from jax.experimental import pallas as pl
from jax.experimental.pallas import tpu as pltpu
from jax.experimental.pallas import tpu_sc as plsc

f32 = jnp.float32
i32 = jnp.int32

N = 10000
E = 320000
IND = 128
OUTD = 64
EDGED = 16
H1 = 32

NC = 2
NS = 16
NW = NC * NS
EPW = E // NW
CH = 80
NCHUNK = EPW // CH
NPAD = 10240
RPT = NPAD // NS

BE = 4000
GE = E // BE
BN = 2000
GN = N // BN


def _ln_k(x, g, b, eps=1e-6):
    m = jnp.mean(x, axis=-1, keepdims=True)
    v = jnp.mean((x - m) ** 2, axis=-1, keepdims=True)
    return (x - m) * lax.rsqrt(v + eps) * g + b


def _full(shape):
    return pl.BlockSpec(shape, lambda i: tuple(0 for _ in shape))


def _edge_enc_body(ea, ge, be, W1t, b1, W2t, b2, W3t, b3, Wc1t, bc1, Wc2t,
                   bc2, out):
    a = ea[...]
    h = _ln_k(a, ge[...], be[...])
    h = jnp.maximum(h @ W1t[...] + b1[...], 0.0)
    h = jnp.maximum(h @ W2t[...] + b2[...], 0.0)
    enc = h @ W3t[...] + b3[...]
    c = jnp.maximum(a @ Wc1t[...] + bc1[...], 0.0)
    w = jax.nn.sigmoid(c @ Wc2t[...] + bc2[...])
    out[...] = enc * w


def _edge_enc(ea, *ws):
    specs = [pl.BlockSpec((BE, EDGED), lambda i: (i, 0))]
    specs += [_full(w.shape) for w in ws]
    return pl.pallas_call(
        _edge_enc_body,
        grid=(GE,),
        in_specs=specs,
        out_specs=pl.BlockSpec((BE, OUTD), lambda i: (i, 0)),
        out_shape=jax.ShapeDtypeStruct((E, OUTD), f32),
    )(ea, *ws)


def _node_body(x_ref, dummy, g0, b0, Adt, Ast, Wst, bs, Wgt, bg,
               pd_ref, ps_ref, gate_ref, gskip_ref):
    x = x_ref[...]
    bad = x[:, 0:1] == -999.0
    x = jnp.where(bad, dummy[...], x)
    xn = _ln_k(x, g0[...], b0[...])
    zpad = jnp.zeros((BN, IND - H1), f32)
    pd_ref[...] = jnp.concatenate([xn @ Adt[...], zpad], axis=1)
    ps_ref[...] = jnp.concatenate([xn @ Ast[...], zpad], axis=1)
    skip = xn @ Wst[...] + bs[...]
    gate = jax.nn.sigmoid(skip @ Wgt[...] + bg[...])
    gate_ref[...] = gate
    gskip_ref[...] = gate * skip


def _node(x, *ws):
    specs = [pl.BlockSpec((BN, IND), lambda i: (i, 0))]
    specs += [_full(w.shape) for w in ws]
    return pl.pallas_call(
        _node_body,
        grid=(GN,),
        in_specs=specs,
        out_specs=[
            pl.BlockSpec((BN, IND), lambda i: (i, 0)),
            pl.BlockSpec((BN, IND), lambda i: (i, 0)),
            pl.BlockSpec((BN, OUTD), lambda i: (i, 0)),
            pl.BlockSpec((BN, OUTD), lambda i: (i, 0)),
        ],
        out_shape=[
            jax.ShapeDtypeStruct((N, IND), f32),
            jax.ShapeDtypeStruct((N, IND), f32),
            jax.ShapeDtypeStruct((N, OUTD), f32),
            jax.ShapeDtypeStruct((N, OUTD), f32),
        ],
    )(x, *ws)


def _mlp1_body(pre, ee, Aet, b1a, W1bt, b1b, W1ct, b1c, out):
    m = jnp.maximum(pre[...][:, :H1] + ee[...] @ Aet[...] + b1a[...], 0.0)
    m = jnp.maximum(m @ W1bt[...] + b1b[...], 0.0)
    m = m @ W1ct[...] + b1c[...]
    colid = lax.broadcasted_iota(i32, (BE, IND - H1), 1)
    aug = jnp.where(colid == 0, 1.0, 0.0).astype(f32)
    out[...] = jnp.concatenate([m, aug], axis=1)


def _mlp1(pre, ee, *ws):
    specs = [
        pl.BlockSpec((BE, IND), lambda i: (i, 0)),
        pl.BlockSpec((BE, OUTD), lambda i: (i, 0)),
    ]
    specs += [_full(w.shape) for w in ws]
    return pl.pallas_call(
        _mlp1_body,
        grid=(GE,),
        in_specs=specs,
        out_specs=pl.BlockSpec((BE, IND), lambda i: (i, 0)),
        out_shape=jax.ShapeDtypeStruct((E, IND), f32),
    )(pre, ee, *ws)


def _x1_body(pa, pb, g1, b1, Bdt, Bst, qd_ref, qs_ref, invd_ref):
    s = pa[0] + pb[0]
    cnt = s[:, H1:H1 + 1]
    invd = 1.0 / jnp.maximum(cnt, 1.0)
    z = _ln_k(s[:, :H1] * invd, g1[...], b1[...])
    z = jnp.where(z >= 0.0, z, 0.01 * z)
    zpad = jnp.zeros((BN, IND - OUTD), f32)
    qd_ref[...] = jnp.concatenate([z @ Bdt[...], zpad], axis=1)
    qs_ref[...] = jnp.concatenate([z @ Bst[...], zpad], axis=1)
    invd_ref[...] = invd


def _x1(s1, g1, b1, Bdt, Bst):
    return pl.pallas_call(
        _x1_body,
        grid=(GN,),
        in_specs=[
            pl.BlockSpec((1, BN, IND), lambda i: (0, i, 0)),
            pl.BlockSpec((1, BN, IND), lambda i: (1, i, 0)),
            _full(g1.shape),
            _full(b1.shape),
            _full(Bdt.shape),
            _full(Bst.shape),
        ],
        out_specs=[
            pl.BlockSpec((BN, IND), lambda i: (i, 0)),
            pl.BlockSpec((BN, IND), lambda i: (i, 0)),
            pl.BlockSpec((BN, 1), lambda i: (i, 0)),
        ],
        out_shape=[
            jax.ShapeDtypeStruct((N, IND), f32),
            jax.ShapeDtypeStruct((N, IND), f32),
            jax.ShapeDtypeStruct((N, 1), f32),
        ],
    )(s1, s1, g1, b1, Bdt, Bst)


def _mlp2_body(pre, ee, Bet, b2a, W2bt, b2b, W2ct, b2c, out):
    e = ee[...]
    m = jnp.maximum(pre[...][:, :OUTD] + e @ Bet[...] + b2a[...], 0.0)
    m = jnp.maximum(m @ W2bt[...] + b2b[...], 0.0)
    m = m @ W2ct[...] + b2c[...]
    out[...] = jnp.concatenate([m, e], axis=1)


def _mlp2(pre, ee, *ws):
    specs = [
        pl.BlockSpec((BE, IND), lambda i: (i, 0)),
        pl.BlockSpec((BE, OUTD), lambda i: (i, 0)),
    ]
    specs += [_full(w.shape) for w in ws]
    return pl.pallas_call(
        _mlp2_body,
        grid=(GE,),
        in_specs=specs,
        out_specs=pl.BlockSpec((BE, IND), lambda i: (i, 0)),
        out_shape=jax.ShapeDtypeStruct((E, IND), f32),
    )(pre, ee, *ws)


def _final_body(p2a, p2b, invd, gate, gskip, g2, b2,
                Wp1t, bp1, Wp2t, bp2, Wp3t, bp3, xfc_ref, probs_ref):
    inv = invd[...]
    s = (p2a[0] + p2b[0]) * inv
    x2 = _ln_k(s[:, :OUTD], g2[...], b2[...])
    x2 = jnp.maximum(x2, 0.0)
    efm = s[:, OUTD:]
    g = gate[...]
    xf = gskip[...] + (1.0 - g) * x2
    xfc = jnp.concatenate([xf, efm], axis=1)
    xfc_ref[...] = xfc
    h = xfc @ Wp1t[...] + bp1[...]
    h = jnp.where(h > 0.0, h, jnp.exp(h) - 1.0)
    h = h @ Wp2t[...] + bp2[...]
    h = jnp.where(h > 0.0, h, jnp.exp(h) - 1.0)
    probs_ref[...] = h @ Wp3t[...] + bp3[...]


def _final(s2, invd, gate, gskip, *ws):
    specs = [
        pl.BlockSpec((1, BN, IND), lambda i: (0, i, 0)),
        pl.BlockSpec((1, BN, IND), lambda i: (1, i, 0)),
        pl.BlockSpec((BN, 1), lambda i: (i, 0)),
        pl.BlockSpec((BN, OUTD), lambda i: (i, 0)),
        pl.BlockSpec((BN, OUTD), lambda i: (i, 0)),
    ]
    specs += [_full(w.shape) for w in ws]
    return pl.pallas_call(
        _final_body,
        grid=(GN,),
        in_specs=specs,
        out_specs=[
            pl.BlockSpec((BN, 2 * OUTD), lambda i: (i, 0)),
            pl.BlockSpec((BN, 1), lambda i: (i, 0)),
        ],
        out_shape=[
            jax.ShapeDtypeStruct((N, 2 * OUTD), f32),
            jax.ShapeDtypeStruct((N, 1), f32),
        ],
    )(s2, s2, invd, gate, gskip, *ws)


_sc_mesh = plsc.VectorSubcoreMesh(
    core_axis_name="c", subcore_axis_name="s", num_cores=NC, num_subcores=NS)


NB = 5
GRP = NCHUNK // NB


@functools.partial(
    pl.kernel,
    out_type=jax.ShapeDtypeStruct((E, IND), f32),
    mesh=_sc_mesh,
    scratch_types=([pltpu.VMEM((CH,), i32)] * (2 * NB)
                   + [pltpu.VMEM((CH, IND), f32)] * NB
                   + [pltpu.SemaphoreType.DMA] * 3),
)
def _sc_gather_add(ta, tb, ia, ib, out, *scr):
    idxa = scr[:NB]
    idxb = scr[NB:2 * NB]
    rows = scr[2 * NB:3 * NB]
    semi, semg, semw = scr[3 * NB:]
    c = lax.axis_index("c")
    s = lax.axis_index("s")
    wid = c * NS + s

    def group(g, carry):
        bases = [wid * EPW + (g * NB + b) * CH for b in range(NB)]
        cps = []
        for b in range(NB):
            cps.append(pltpu.async_copy(ia.at[pl.ds(bases[b], CH)],
                                        idxa[b], semi))
            cps.append(pltpu.async_copy(ib.at[pl.ds(bases[b], CH)],
                                        idxb[b], semi))
        for cp in cps:
            cp.wait()
        cps = [pltpu.async_copy(ta.at[idxa[b]], rows[b], semg)
               for b in range(NB)]
        for cp in cps:
            cp.wait()
        cps = [pltpu.async_copy(tb.at[idxb[b]], rows[b], semg, add=True)
               for b in range(NB)]
        for cp in cps:
            cp.wait()
        cps = [pltpu.async_copy(rows[b], out.at[pl.ds(bases[b], CH)], semw)
               for b in range(NB)]
        for cp in cps:
            cp.wait()
        return carry

    lax.fori_loop(0, GRP, group, 0)


NBS = 4
GRPS = NCHUNK // NBS


@functools.partial(
    pl.kernel,
    out_type=jax.ShapeDtypeStruct((NC, NPAD, IND), f32),
    mesh=_sc_mesh,
    scratch_types=([pltpu.VMEM((CH,), i32)] * NBS
                   + [pltpu.VMEM((CH, IND), f32)] * NBS
                   + [pltpu.VMEM_SHARED((NPAD, IND), f32)]
                   + [pltpu.SemaphoreType.DMA] * 2),
)
def _sc_scatter128(vals, dsti, zer, out, *scr):
    idx = scr[:NBS]
    rows = scr[NBS:2 * NBS]
    acc = scr[2 * NBS]
    semi, sems = scr[2 * NBS + 1:]
    c = lax.axis_index("c")
    s = lax.axis_index("s")
    r0 = s * RPT
    pltpu.sync_copy(zer.at[pl.ds(r0, RPT)], acc.at[pl.ds(r0, RPT)])
    plsc.subcore_barrier()
    wid = c * NS + s

    def group(g, carry):
        bases = [wid * EPW + (g * NBS + b) * CH for b in range(NBS)]
        cps = []
        for b in range(NBS):
            cps.append(pltpu.async_copy(dsti.at[pl.ds(bases[b], CH)],
                                        idx[b], semi))
            cps.append(pltpu.async_copy(vals.at[pl.ds(bases[b], CH)],
                                        rows[b], semi))
        for cp in cps:
            cp.wait()
        cps = [pltpu.async_copy(rows[b], acc.at[idx[b]], sems, add=True)
               for b in range(NBS)]
        for cp in cps:
            cp.wait()
        return carry

    lax.fori_loop(0, GRPS, group, 0)
    tail = wid * EPW + GRPS * NBS * CH
    pltpu.sync_copy(dsti.at[pl.ds(tail, CH)], idx[0])
    pltpu.sync_copy(vals.at[pl.ds(tail, CH)], rows[0])
    pltpu.sync_copy(rows[0], acc.at[idx[0]], add=True)
    plsc.subcore_barrier()
    pltpu.sync_copy(acc.at[pl.ds(r0, RPT)], out.at[c, pl.ds(r0, RPT)])


def kernel(x_in, edge_index, edge_attr, params):
    p = params
    src = edge_index[0, 0]
    dst = edge_index[0, 1]
    x = x_in[0]
    ea = edge_attr[0]

    def r(v):
        return v.reshape(1, -1)

    Adt = p['Wm1a'][:, :IND].T
    Ast = p['Wm1a'][:, IND:2 * IND].T
    Aet = p['Wm1a'][:, 2 * IND:].T
    Bdt = p['Wm2a'][:, :H1].T
    Bst = p['Wm2a'][:, H1:2 * H1].T
    Bet = p['Wm2a'][:, 2 * H1:].T
    zer = jnp.zeros((NPAD, IND), f32)

    e_enc = _edge_enc(ea, r(p['ge']), r(p['be']),
                      p['We1'].T, r(p['be1']), p['We2'].T, r(p['be2']),
                      p['We3'].T, r(p['be3']),
                      p['Wc1'].T, r(p['bc1']), p['Wc2'].T, r(p['bc2']))
    pd, ps, gate, gskip = _node(x, r(p['dummy']), r(p['g0']), r(p['b0']),
                                Adt, Ast,
                                p['Wskip'].T, r(p['bskip']),
                                p['Wg'].T, r(p['bg']))
    m1pre = _sc_gather_add(pd, ps, dst, src)
    m1 = _mlp1(m1pre, e_enc, Aet, r(p['bm1a']),
               p['Wm1b'].T, r(p['bm1b']), p['Wm1c'].T, r(p['bm1c']))
    s1 = _sc_scatter128(m1, dst, zer)
    qd, qs, invd = _x1(s1, r(p['g1']), r(p['b1']), Bdt, Bst)
    m2pre = _sc_gather_add(qd, qs, dst, src)
    m2 = _mlp2(m2pre, e_enc, Bet, r(p['bm2a']),
               p['Wm2b'].T, r(p['bm2b']), p['Wm2c'].T, r(p['bm2c']))
    s2 = _sc_scatter128(m2, dst, zer)
    xfc, probs = _final(s2, invd, gate, gskip,
                        r(p['g2']), r(p['b2']),
                        p['Wp1'].T, r(p['bp1']), p['Wp2'].T, r(p['bp2']),
                        p['Wp3'].T, r(p['bp3']))
    return (xfc[None], probs[None], jnp.zeros((1,), f32))


# speedup vs baseline: 4.8830x; 1.1206x over previous
import functools

import jax
import jax.numpy as jnp
from jax import lax
from jax.experimental import pallas as pl
from jax.experimental.pallas import tpu as pltpu
from jax.experimental.pallas import tpu_sc as plsc

f32 = jnp.float32
i32 = jnp.int32

N = 10000
E = 320000
IND = 128
OUTD = 64
EDGED = 16
H1 = 32

NC = 2
NS = 16
NW = NC * NS
EPW = E // NW
CH = 80
NCHUNK = EPW // CH
NPAD = 10240
RPT = NPAD // NS

BE = 4000
GE = E // BE
BN = 2000
GN = N // BN


def _ln_k(x, g, b, eps=1e-6):
    m = jnp.mean(x, axis=-1, keepdims=True)
    v = jnp.mean((x - m) ** 2, axis=-1, keepdims=True)
    return (x - m) * lax.rsqrt(v + eps) * g + b


def _full(shape):
    return pl.BlockSpec(shape, lambda i: tuple(0 for _ in shape))


def _edge_enc_body(ea, ge, be, W1t, b1, W2t, b2, W3t, b3, Wc1t, bc1, Wc2t,
                   bc2, out):
    a = ea[...]
    h = _ln_k(a, ge[...], be[...])
    h = jnp.maximum(h @ W1t[...] + b1[...], 0.0)
    h = jnp.maximum(h @ W2t[...] + b2[...], 0.0)
    enc = h @ W3t[...] + b3[...]
    c = jnp.maximum(a @ Wc1t[...] + bc1[...], 0.0)
    w = jax.nn.sigmoid(c @ Wc2t[...] + bc2[...])
    out[...] = enc * w


def _edge_enc(ea, *ws):
    specs = [pl.BlockSpec((BE, EDGED), lambda i: (i, 0))]
    specs += [_full(w.shape) for w in ws]
    return pl.pallas_call(
        _edge_enc_body,
        grid=(GE,),
        in_specs=specs,
        out_specs=pl.BlockSpec((BE, OUTD), lambda i: (i, 0)),
        out_shape=jax.ShapeDtypeStruct((E, OUTD), f32),
    )(ea, *ws)


def _node_body(x_ref, dummy, g0, b0, Adt, Ast, Wst, bs, Wgt, bg,
               pd_ref, ps_ref, gate_ref, gskip_ref):
    x = x_ref[...]
    bad = x[:, 0:1] == -999.0
    x = jnp.where(bad, dummy[...], x)
    xn = _ln_k(x, g0[...], b0[...])
    pd_ref[...] = xn @ Adt[...]
    ps_ref[...] = xn @ Ast[...]
    skip = xn @ Wst[...] + bs[...]
    gate = jax.nn.sigmoid(skip @ Wgt[...] + bg[...])
    gate_ref[...] = gate
    gskip_ref[...] = gate * skip


def _node(x, *ws):
    specs = [pl.BlockSpec((BN, IND), lambda i: (i, 0))]
    specs += [_full(w.shape) for w in ws]
    return pl.pallas_call(
        _node_body,
        grid=(GN,),
        in_specs=specs,
        out_specs=[
            pl.BlockSpec((BN, H1), lambda i: (i, 0)),
            pl.BlockSpec((BN, H1), lambda i: (i, 0)),
            pl.BlockSpec((BN, OUTD), lambda i: (i, 0)),
            pl.BlockSpec((BN, OUTD), lambda i: (i, 0)),
        ],
        out_shape=[
            jax.ShapeDtypeStruct((N, H1), f32),
            jax.ShapeDtypeStruct((N, H1), f32),
            jax.ShapeDtypeStruct((N, OUTD), f32),
            jax.ShapeDtypeStruct((N, OUTD), f32),
        ],
    )(x, *ws)


def _mlp1_body(pre, ee, Aet, b1a, W1bt, b1b, W1ct, b1c, out):
    m = jnp.maximum(pre[...][:, :H1] + ee[...] @ Aet[...] + b1a[...], 0.0)
    m = jnp.maximum(m @ W1bt[...] + b1b[...], 0.0)
    m = m @ W1ct[...] + b1c[...]
    colid = lax.broadcasted_iota(i32, (BE, IND - H1), 1)
    aug = jnp.where(colid == 0, 1.0, 0.0).astype(f32)
    out[...] = jnp.concatenate([m, aug], axis=1)


def _mlp1(pre, ee, *ws):
    specs = [
        pl.BlockSpec((BE, IND), lambda i: (i, 0)),
        pl.BlockSpec((BE, OUTD), lambda i: (i, 0)),
    ]
    specs += [_full(w.shape) for w in ws]
    return pl.pallas_call(
        _mlp1_body,
        grid=(GE,),
        in_specs=specs,
        out_specs=pl.BlockSpec((BE, IND), lambda i: (i, 0)),
        out_shape=jax.ShapeDtypeStruct((E, IND), f32),
    )(pre, ee, *ws)


def _x1_body(pa, pb, g1, b1, Bdt, Bst, qd_ref, qs_ref, invd_ref):
    s = pa[0] + pb[0]
    cnt = s[:, H1:H1 + 1]
    invd = 1.0 / jnp.maximum(cnt, 1.0)
    z = _ln_k(s[:, :H1] * invd, g1[...], b1[...])
    z = jnp.where(z >= 0.0, z, 0.01 * z)
    qd_ref[...] = z @ Bdt[...]
    qs_ref[...] = z @ Bst[...]
    invd_ref[...] = invd


def _x1(s1, g1, b1, Bdt, Bst):
    return pl.pallas_call(
        _x1_body,
        grid=(GN,),
        in_specs=[
            pl.BlockSpec((1, BN, IND), lambda i: (0, i, 0)),
            pl.BlockSpec((1, BN, IND), lambda i: (1, i, 0)),
            _full(g1.shape),
            _full(b1.shape),
            _full(Bdt.shape),
            _full(Bst.shape),
        ],
        out_specs=[
            pl.BlockSpec((BN, OUTD), lambda i: (i, 0)),
            pl.BlockSpec((BN, OUTD), lambda i: (i, 0)),
            pl.BlockSpec((BN, 1), lambda i: (i, 0)),
        ],
        out_shape=[
            jax.ShapeDtypeStruct((N, OUTD), f32),
            jax.ShapeDtypeStruct((N, OUTD), f32),
            jax.ShapeDtypeStruct((N, 1), f32),
        ],
    )(s1, s1, g1, b1, Bdt, Bst)


def _mlp2_body(pre, ee, Bet, b2a, W2bt, b2b, W2ct, b2c, out):
    e = ee[...]
    m = jnp.maximum(pre[...][:, :OUTD] + e @ Bet[...] + b2a[...], 0.0)
    m = jnp.maximum(m @ W2bt[...] + b2b[...], 0.0)
    m = m @ W2ct[...] + b2c[...]
    out[...] = jnp.concatenate([m, e], axis=1)


def _mlp2(pre, ee, *ws):
    specs = [
        pl.BlockSpec((BE, IND), lambda i: (i, 0)),
        pl.BlockSpec((BE, OUTD), lambda i: (i, 0)),
    ]
    specs += [_full(w.shape) for w in ws]
    return pl.pallas_call(
        _mlp2_body,
        grid=(GE,),
        in_specs=specs,
        out_specs=pl.BlockSpec((BE, IND), lambda i: (i, 0)),
        out_shape=jax.ShapeDtypeStruct((E, IND), f32),
    )(pre, ee, *ws)


def _final_body(p2a, p2b, invd, gate, gskip, g2, b2,
                Wp1t, bp1, Wp2t, bp2, Wp3t, bp3, xfc_ref, probs_ref):
    inv = invd[...]
    s = (p2a[0] + p2b[0]) * inv
    x2 = _ln_k(s[:, :OUTD], g2[...], b2[...])
    x2 = jnp.maximum(x2, 0.0)
    efm = s[:, OUTD:]
    g = gate[...]
    xf = gskip[...] + (1.0 - g) * x2
    xfc = jnp.concatenate([xf, efm], axis=1)
    xfc_ref[...] = xfc
    h = xfc @ Wp1t[...] + bp1[...]
    h = jnp.where(h > 0.0, h, jnp.exp(h) - 1.0)
    h = h @ Wp2t[...] + bp2[...]
    h = jnp.where(h > 0.0, h, jnp.exp(h) - 1.0)
    probs_ref[...] = h @ Wp3t[...] + bp3[...]


def _final(s2, invd, gate, gskip, *ws):
    specs = [
        pl.BlockSpec((1, BN, IND), lambda i: (0, i, 0)),
        pl.BlockSpec((1, BN, IND), lambda i: (1, i, 0)),
        pl.BlockSpec((BN, 1), lambda i: (i, 0)),
        pl.BlockSpec((BN, OUTD), lambda i: (i, 0)),
        pl.BlockSpec((BN, OUTD), lambda i: (i, 0)),
    ]
    specs += [_full(w.shape) for w in ws]
    return pl.pallas_call(
        _final_body,
        grid=(GN,),
        in_specs=specs,
        out_specs=[
            pl.BlockSpec((BN, 2 * OUTD), lambda i: (i, 0)),
            pl.BlockSpec((BN, 1), lambda i: (i, 0)),
        ],
        out_shape=[
            jax.ShapeDtypeStruct((N, 2 * OUTD), f32),
            jax.ShapeDtypeStruct((N, 1), f32),
        ],
    )(s2, s2, invd, gate, gskip, *ws)


_sc_mesh = plsc.VectorSubcoreMesh(
    core_axis_name="c", subcore_axis_name="s", num_cores=NC, num_subcores=NS)


NB = 5
GRP = NCHUNK // NB


def _make_gather_add(W):
    @functools.partial(
        pl.kernel,
        out_type=jax.ShapeDtypeStruct((E, IND), f32),
        mesh=_sc_mesh,
        scratch_types=([pltpu.VMEM((CH,), i32)] * (2 * NB)
                       + [pltpu.VMEM((CH, W), f32)] * NB
                       + [pltpu.SemaphoreType.DMA] * 3),
        compiler_params=pltpu.CompilerParams(use_tc_tiling_on_sc=False),
    )
    def gather_add(ta, tb, ia, ib, out, *scr):
        idxa = scr[:NB]
        idxb = scr[NB:2 * NB]
        rows = scr[2 * NB:3 * NB]
        semi, semg, semw = scr[3 * NB:]
        c = lax.axis_index("c")
        s = lax.axis_index("s")
        wid = c * NS + s

        def group(g, carry):
            bases = [wid * EPW + (g * NB + b) * CH for b in range(NB)]
            cps = []
            for b in range(NB):
                cps.append(pltpu.async_copy(ia.at[pl.ds(bases[b], CH)],
                                            idxa[b], semi))
                cps.append(pltpu.async_copy(ib.at[pl.ds(bases[b], CH)],
                                            idxb[b], semi))
            for cp in cps:
                cp.wait()
            cps = [pltpu.async_copy(ta.at[idxa[b]], rows[b], semg)
                   for b in range(NB)]
            for cp in cps:
                cp.wait()
            cps = [pltpu.async_copy(tb.at[idxb[b]], rows[b], semg, add=True)
                   for b in range(NB)]
            for cp in cps:
                cp.wait()
            cps = [pltpu.async_copy(
                rows[b], out.at[pl.ds(bases[b], CH), pl.ds(0, W)], semw)
                for b in range(NB)]
            for cp in cps:
                cp.wait()
            return carry

        lax.fori_loop(0, GRP, group, 0)

    return gather_add


_sc_gather_add32 = _make_gather_add(H1)
_sc_gather_add64 = _make_gather_add(OUTD)


NBS = 4
GRPS = NCHUNK // NBS


@functools.partial(
    pl.kernel,
    out_type=jax.ShapeDtypeStruct((NC, NPAD, IND), f32),
    mesh=_sc_mesh,
    scratch_types=([pltpu.VMEM((CH,), i32)] * NBS
                   + [pltpu.VMEM((CH, IND), f32)] * NBS
                   + [pltpu.VMEM_SHARED((NPAD, IND), f32)]
                   + [pltpu.SemaphoreType.DMA] * 2),
)
def _sc_scatter128(vals, dsti, zer, out, *scr):
    idx = scr[:NBS]
    rows = scr[NBS:2 * NBS]
    acc = scr[2 * NBS]
    semi, sems = scr[2 * NBS + 1:]
    c = lax.axis_index("c")
    s = lax.axis_index("s")
    r0 = s * RPT
    pltpu.sync_copy(zer.at[pl.ds(r0, RPT)], acc.at[pl.ds(r0, RPT)])
    plsc.subcore_barrier()
    wid = c * NS + s

    def group(g, carry):
        bases = [wid * EPW + (g * NBS + b) * CH for b in range(NBS)]
        cps = []
        for b in range(NBS):
            cps.append(pltpu.async_copy(dsti.at[pl.ds(bases[b], CH)],
                                        idx[b], semi))
            cps.append(pltpu.async_copy(vals.at[pl.ds(bases[b], CH)],
                                        rows[b], semi))
        for cp in cps:
            cp.wait()
        cps = [pltpu.async_copy(rows[b], acc.at[idx[b]], sems, add=True)
               for b in range(NBS)]
        for cp in cps:
            cp.wait()
        return carry

    lax.fori_loop(0, GRPS, group, 0)
    tail = wid * EPW + GRPS * NBS * CH
    pltpu.sync_copy(dsti.at[pl.ds(tail, CH)], idx[0])
    pltpu.sync_copy(vals.at[pl.ds(tail, CH)], rows[0])
    pltpu.sync_copy(rows[0], acc.at[idx[0]], add=True)
    plsc.subcore_barrier()
    pltpu.sync_copy(acc.at[pl.ds(r0, RPT)], out.at[c, pl.ds(r0, RPT)])


def kernel(x_in, edge_index, edge_attr, params):
    p = params
    src = edge_index[0, 0]
    dst = edge_index[0, 1]
    x = x_in[0]
    ea = edge_attr[0]

    def r(v):
        return v.reshape(1, -1)

    Adt = p['Wm1a'][:, :IND].T
    Ast = p['Wm1a'][:, IND:2 * IND].T
    Aet = p['Wm1a'][:, 2 * IND:].T
    Bdt = p['Wm2a'][:, :H1].T
    Bst = p['Wm2a'][:, H1:2 * H1].T
    Bet = p['Wm2a'][:, 2 * H1:].T
    zer = jnp.zeros((NPAD, IND), f32)

    e_enc = _edge_enc(ea, r(p['ge']), r(p['be']),
                      p['We1'].T, r(p['be1']), p['We2'].T, r(p['be2']),
                      p['We3'].T, r(p['be3']),
                      p['Wc1'].T, r(p['bc1']), p['Wc2'].T, r(p['bc2']))
    pd, ps, gate, gskip = _node(x, r(p['dummy']), r(p['g0']), r(p['b0']),
                                Adt, Ast,
                                p['Wskip'].T, r(p['bskip']),
                                p['Wg'].T, r(p['bg']))
    m1pre = _sc_gather_add32(pd, ps, dst, src)
    m1 = _mlp1(m1pre, e_enc, Aet, r(p['bm1a']),
               p['Wm1b'].T, r(p['bm1b']), p['Wm1c'].T, r(p['bm1c']))
    s1 = _sc_scatter128(m1, dst, zer)
    qd, qs, invd = _x1(s1, r(p['g1']), r(p['b1']), Bdt, Bst)
    m2pre = _sc_gather_add64(qd, qs, dst, src)
    m2 = _mlp2(m2pre, e_enc, Bet, r(p['bm2a']),
               p['Wm2b'].T, r(p['bm2b']), p['Wm2c'].T, r(p['bm2c']))
    s2 = _sc_scatter128(m2, dst, zer)
    xfc, probs = _final(s2, invd, gate, gskip,
                        r(p['g2']), r(p['b2']),
                        p['Wp1'].T, r(p['bp1']), p['Wp2'].T, r(p['bp2']),
                        p['Wp3'].T, r(p['bp3']))
    return (xfc[None], probs[None], jnp.zeros((1,), f32))


# speedup vs baseline: 4.9849x; 1.0209x over previous
import functools

import jax
import jax.numpy as jnp
from jax import lax
from jax.experimental import pallas as pl
from jax.experimental.pallas import tpu as pltpu
from jax.experimental.pallas import tpu_sc as plsc

f32 = jnp.float32
i32 = jnp.int32

N = 10000
E = 320000
IND = 128
OUTD = 64
EDGED = 16
H1 = 32

NC = 2
NS = 16
NW = NC * NS
EPW = E // NW
CH = 80
NCHUNK = EPW // CH
NPAD = 10240
RPT = NPAD // NS

BE = 4000
GE = E // BE
BN = 2000
GN = N // BN


def _ln_k(x, g, b, eps=1e-6):
    m = jnp.mean(x, axis=-1, keepdims=True)
    v = jnp.mean((x - m) ** 2, axis=-1, keepdims=True)
    return (x - m) * lax.rsqrt(v + eps) * g + b


def _full(shape):
    return pl.BlockSpec(shape, lambda i: tuple(0 for _ in shape))


def _edge_enc_body(ea, ge, be, W1t, b1, W2t, b2, W3t, b3, Wc1t, bc1, Wc2t,
                   bc2, out):
    a = ea[...]
    h = _ln_k(a, ge[...], be[...])
    h = jnp.maximum(h @ W1t[...] + b1[...], 0.0)
    h = jnp.maximum(h @ W2t[...] + b2[...], 0.0)
    enc = h @ W3t[...] + b3[...]
    c = jnp.maximum(a @ Wc1t[...] + bc1[...], 0.0)
    w = jax.nn.sigmoid(c @ Wc2t[...] + bc2[...])
    out[...] = enc * w


def _edge_enc(ea, *ws):
    specs = [pl.BlockSpec((BE, EDGED), lambda i: (i, 0))]
    specs += [_full(w.shape) for w in ws]
    return pl.pallas_call(
        _edge_enc_body,
        grid=(GE,),
        in_specs=specs,
        out_specs=pl.BlockSpec((BE, OUTD), lambda i: (i, 0)),
        out_shape=jax.ShapeDtypeStruct((E, OUTD), f32),
    )(ea, *ws)


def _node_body(x_ref, dummy, g0, b0, Adt, Ast, Wst, bs, Wgt, bg,
               pd_ref, ps_ref, gate_ref, gskip_ref):
    x = x_ref[...]
    bad = x[:, 0:1] == -999.0
    x = jnp.where(bad, dummy[...], x)
    xn = _ln_k(x, g0[...], b0[...])
    pd_ref[...] = xn @ Adt[...]
    ps_ref[...] = xn @ Ast[...]
    skip = xn @ Wst[...] + bs[...]
    gate = jax.nn.sigmoid(skip @ Wgt[...] + bg[...])
    gate_ref[...] = gate
    gskip_ref[...] = gate * skip


def _node(x, *ws):
    specs = [pl.BlockSpec((BN, IND), lambda i: (i, 0))]
    specs += [_full(w.shape) for w in ws]
    return pl.pallas_call(
        _node_body,
        grid=(GN,),
        in_specs=specs,
        out_specs=[
            pl.BlockSpec((BN, H1), lambda i: (i, 0)),
            pl.BlockSpec((BN, H1), lambda i: (i, 0)),
            pl.BlockSpec((BN, OUTD), lambda i: (i, 0)),
            pl.BlockSpec((BN, OUTD), lambda i: (i, 0)),
        ],
        out_shape=[
            jax.ShapeDtypeStruct((N, H1), f32),
            jax.ShapeDtypeStruct((N, H1), f32),
            jax.ShapeDtypeStruct((N, OUTD), f32),
            jax.ShapeDtypeStruct((N, OUTD), f32),
        ],
    )(x, *ws)


def _mlp1_body(pre, ee, Aet, b1a, W1bt, b1b, W1ct, b1c, out):
    m = jnp.maximum(pre[...][:, :H1] + ee[...] @ Aet[...] + b1a[...], 0.0)
    m = jnp.maximum(m @ W1bt[...] + b1b[...], 0.0)
    m = m @ W1ct[...] + b1c[...]
    colid = lax.broadcasted_iota(i32, (BE, IND - H1), 1)
    aug = jnp.where(colid == 0, 1.0, 0.0).astype(f32)
    out[...] = jnp.concatenate([m, aug], axis=1)


def _mlp1(pre, ee, *ws):
    specs = [
        pl.BlockSpec((BE, IND), lambda i: (i, 0)),
        pl.BlockSpec((BE, OUTD), lambda i: (i, 0)),
    ]
    specs += [_full(w.shape) for w in ws]
    return pl.pallas_call(
        _mlp1_body,
        grid=(GE,),
        in_specs=specs,
        out_specs=pl.BlockSpec((BE, IND), lambda i: (i, 0)),
        out_shape=jax.ShapeDtypeStruct((E, IND), f32),
    )(pre, ee, *ws)


def _x1_body(pa, pb, g1, b1, Bdt, Bst, qd_ref, qs_ref, invd_ref):
    s = pa[0] + pb[0]
    cnt = s[:, H1:H1 + 1]
    invd = 1.0 / jnp.maximum(cnt, 1.0)
    z = _ln_k(s[:, :H1] * invd, g1[...], b1[...])
    z = jnp.where(z >= 0.0, z, 0.01 * z)
    qd_ref[...] = z @ Bdt[...]
    qs_ref[...] = z @ Bst[...]
    invd_ref[...] = invd


def _x1(s1, g1, b1, Bdt, Bst):
    return pl.pallas_call(
        _x1_body,
        grid=(GN,),
        in_specs=[
            pl.BlockSpec((1, BN, IND), lambda i: (0, i, 0)),
            pl.BlockSpec((1, BN, IND), lambda i: (1, i, 0)),
            _full(g1.shape),
            _full(b1.shape),
            _full(Bdt.shape),
            _full(Bst.shape),
        ],
        out_specs=[
            pl.BlockSpec((BN, OUTD), lambda i: (i, 0)),
            pl.BlockSpec((BN, OUTD), lambda i: (i, 0)),
            pl.BlockSpec((BN, 1), lambda i: (i, 0)),
        ],
        out_shape=[
            jax.ShapeDtypeStruct((N, OUTD), f32),
            jax.ShapeDtypeStruct((N, OUTD), f32),
            jax.ShapeDtypeStruct((N, 1), f32),
        ],
    )(s1, s1, g1, b1, Bdt, Bst)


def _mlp2_body(pre, ee, Bet, b2a, W2bt, b2b, W2ct, b2c, out):
    e = ee[...]
    m = jnp.maximum(pre[...][:, :OUTD] + e @ Bet[...] + b2a[...], 0.0)
    m = jnp.maximum(m @ W2bt[...] + b2b[...], 0.0)
    m = m @ W2ct[...] + b2c[...]
    out[...] = jnp.concatenate([m, e], axis=1)


def _mlp2(pre, ee, *ws):
    specs = [
        pl.BlockSpec((BE, IND), lambda i: (i, 0)),
        pl.BlockSpec((BE, OUTD), lambda i: (i, 0)),
    ]
    specs += [_full(w.shape) for w in ws]
    return pl.pallas_call(
        _mlp2_body,
        grid=(GE,),
        in_specs=specs,
        out_specs=pl.BlockSpec((BE, IND), lambda i: (i, 0)),
        out_shape=jax.ShapeDtypeStruct((E, IND), f32),
    )(pre, ee, *ws)


def _final_body(p2a, p2b, invd, gate, gskip, g2, b2,
                Wp1t, bp1, Wp2t, bp2, Wp3t, bp3, xfc_ref, probs_ref):
    inv = invd[...]
    s = (p2a[0] + p2b[0]) * inv
    x2 = _ln_k(s[:, :OUTD], g2[...], b2[...])
    x2 = jnp.maximum(x2, 0.0)
    efm = s[:, OUTD:]
    g = gate[...]
    xf = gskip[...] + (1.0 - g) * x2
    xfc = jnp.concatenate([xf, efm], axis=1)
    xfc_ref[...] = xfc
    h = xfc @ Wp1t[...] + bp1[...]
    h = jnp.where(h > 0.0, h, jnp.exp(h) - 1.0)
    h = h @ Wp2t[...] + bp2[...]
    h = jnp.where(h > 0.0, h, jnp.exp(h) - 1.0)
    probs_ref[...] = h @ Wp3t[...] + bp3[...]


def _final(s2, invd, gate, gskip, *ws):
    specs = [
        pl.BlockSpec((1, BN, IND), lambda i: (0, i, 0)),
        pl.BlockSpec((1, BN, IND), lambda i: (1, i, 0)),
        pl.BlockSpec((BN, 1), lambda i: (i, 0)),
        pl.BlockSpec((BN, OUTD), lambda i: (i, 0)),
        pl.BlockSpec((BN, OUTD), lambda i: (i, 0)),
    ]
    specs += [_full(w.shape) for w in ws]
    return pl.pallas_call(
        _final_body,
        grid=(GN,),
        in_specs=specs,
        out_specs=[
            pl.BlockSpec((BN, 2 * OUTD), lambda i: (i, 0)),
            pl.BlockSpec((BN, 1), lambda i: (i, 0)),
        ],
        out_shape=[
            jax.ShapeDtypeStruct((N, 2 * OUTD), f32),
            jax.ShapeDtypeStruct((N, 1), f32),
        ],
    )(s2, s2, invd, gate, gskip, *ws)


_sc_mesh = plsc.VectorSubcoreMesh(
    core_axis_name="c", subcore_axis_name="s", num_cores=NC, num_subcores=NS)


NB = 5
GRP = NCHUNK // NB
CHG = 200
GRPG = EPW // (CHG * NB)


def _make_gather_add(W):
    @functools.partial(
        pl.kernel,
        out_type=jax.ShapeDtypeStruct((E, IND), f32),
        mesh=_sc_mesh,
        scratch_types=([pltpu.VMEM((CHG,), i32)] * (2 * NB)
                       + [pltpu.VMEM((CHG, W), f32)] * NB
                       + [pltpu.SemaphoreType.DMA] * 3),
        compiler_params=pltpu.CompilerParams(use_tc_tiling_on_sc=False),
    )
    def gather_add(ta, tb, ia, ib, out, *scr):
        idxa = scr[:NB]
        idxb = scr[NB:2 * NB]
        rows = scr[2 * NB:3 * NB]
        semi, semg, semw = scr[3 * NB:]
        c = lax.axis_index("c")
        s = lax.axis_index("s")
        wid = c * NS + s

        def group(g, carry):
            bases = [wid * EPW + (g * NB + b) * CHG for b in range(NB)]
            cps = []
            for b in range(NB):
                cps.append(pltpu.async_copy(ia.at[pl.ds(bases[b], CHG)],
                                            idxa[b], semi))
                cps.append(pltpu.async_copy(ib.at[pl.ds(bases[b], CHG)],
                                            idxb[b], semi))
            for cp in cps:
                cp.wait()
            cps = [pltpu.async_copy(ta.at[idxa[b]], rows[b], semg)
                   for b in range(NB)]
            for cp in cps:
                cp.wait()
            cps = [pltpu.async_copy(tb.at[idxb[b]], rows[b], semg, add=True)
                   for b in range(NB)]
            for cp in cps:
                cp.wait()
            cps = [pltpu.async_copy(
                rows[b], out.at[pl.ds(bases[b], CHG), pl.ds(0, W)], semw)
                for b in range(NB)]
            for cp in cps:
                cp.wait()
            return carry

        lax.fori_loop(0, GRPG, group, 0)

    return gather_add


_sc_gather_add32 = _make_gather_add(H1)
_sc_gather_add64 = _make_gather_add(OUTD)


NBS = 4
GRPS = NCHUNK // NBS


@functools.partial(
    pl.kernel,
    out_type=jax.ShapeDtypeStruct((NC, NPAD, IND), f32),
    mesh=_sc_mesh,
    scratch_types=([pltpu.VMEM((CH,), i32)] * NBS
                   + [pltpu.VMEM((CH, IND), f32)] * NBS
                   + [pltpu.VMEM_SHARED((NPAD, IND), f32)]
                   + [pltpu.SemaphoreType.DMA] * 2),
)
def _sc_scatter128(vals, dsti, zer, out, *scr):
    idx = scr[:NBS]
    rows = scr[NBS:2 * NBS]
    acc = scr[2 * NBS]
    semi, sems = scr[2 * NBS + 1:]
    c = lax.axis_index("c")
    s = lax.axis_index("s")
    r0 = s * RPT
    pltpu.sync_copy(zer.at[pl.ds(r0, RPT)], acc.at[pl.ds(r0, RPT)])
    plsc.subcore_barrier()
    wid = c * NS + s

    def group(g, carry):
        bases = [wid * EPW + (g * NBS + b) * CH for b in range(NBS)]
        cps = []
        for b in range(NBS):
            cps.append(pltpu.async_copy(dsti.at[pl.ds(bases[b], CH)],
                                        idx[b], semi))
            cps.append(pltpu.async_copy(vals.at[pl.ds(bases[b], CH)],
                                        rows[b], semi))
        for cp in cps:
            cp.wait()
        cps = [pltpu.async_copy(rows[b], acc.at[idx[b]], sems, add=True)
               for b in range(NBS)]
        for cp in cps:
            cp.wait()
        return carry

    lax.fori_loop(0, GRPS, group, 0)
    tail = wid * EPW + GRPS * NBS * CH
    pltpu.sync_copy(dsti.at[pl.ds(tail, CH)], idx[0])
    pltpu.sync_copy(vals.at[pl.ds(tail, CH)], rows[0])
    pltpu.sync_copy(rows[0], acc.at[idx[0]], add=True)
    plsc.subcore_barrier()
    pltpu.sync_copy(acc.at[pl.ds(r0, RPT)], out.at[c, pl.ds(r0, RPT)])


def kernel(x_in, edge_index, edge_attr, params):
    p = params
    src = edge_index[0, 0]
    dst = edge_index[0, 1]
    x = x_in[0]
    ea = edge_attr[0]

    def r(v):
        return v.reshape(1, -1)

    Adt = p['Wm1a'][:, :IND].T
    Ast = p['Wm1a'][:, IND:2 * IND].T
    Aet = p['Wm1a'][:, 2 * IND:].T
    Bdt = p['Wm2a'][:, :H1].T
    Bst = p['Wm2a'][:, H1:2 * H1].T
    Bet = p['Wm2a'][:, 2 * H1:].T
    zer = jnp.zeros((NPAD, IND), f32)

    pd, ps, gate, gskip = _node(x, r(p['dummy']), r(p['g0']), r(p['b0']),
                                Adt, Ast,
                                p['Wskip'].T, r(p['bskip']),
                                p['Wg'].T, r(p['bg']))
    m1pre = _sc_gather_add32(pd, ps, dst, src)
    e_enc = _edge_enc(ea, r(p['ge']), r(p['be']),
                      p['We1'].T, r(p['be1']), p['We2'].T, r(p['be2']),
                      p['We3'].T, r(p['be3']),
                      p['Wc1'].T, r(p['bc1']), p['Wc2'].T, r(p['bc2']))
    m1 = _mlp1(m1pre, e_enc, Aet, r(p['bm1a']),
               p['Wm1b'].T, r(p['bm1b']), p['Wm1c'].T, r(p['bm1c']))
    s1 = _sc_scatter128(m1, dst, zer)
    qd, qs, invd = _x1(s1, r(p['g1']), r(p['b1']), Bdt, Bst)
    m2pre = _sc_gather_add64(qd, qs, dst, src)
    m2 = _mlp2(m2pre, e_enc, Bet, r(p['bm2a']),
               p['Wm2b'].T, r(p['bm2b']), p['Wm2c'].T, r(p['bm2c']))
    s2 = _sc_scatter128(m2, dst, zer)
    xfc, probs = _final(s2, invd, gate, gskip,
                        r(p['g2']), r(p['b2']),
                        p['Wp1'].T, r(p['bp1']), p['Wp2'].T, r(p['bp2']),
                        p['Wp3'].T, r(p['bp3']))
    return (xfc[None], probs[None], jnp.zeros((1,), f32))


# speedup vs baseline: 5.7160x; 1.1467x over previous
import functools

import jax
import jax.numpy as jnp
from jax import lax
from jax.experimental import pallas as pl
from jax.experimental.pallas import tpu as pltpu
from jax.experimental.pallas import tpu_sc as plsc

f32 = jnp.float32
i32 = jnp.int32

N = 10000
E = 320000
IND = 128
OUTD = 64
EDGED = 16
H1 = 32

NC = 2
NS = 16
NW = NC * NS
EPW = E // NW
CH = 80
NCHUNK = EPW // CH
NPAD = 10240
RPT = NPAD // NS

BE = 4000
GE = E // BE
BN = 2000
GN = N // BN


def _ln_k(x, g, b, eps=1e-6):
    m = jnp.mean(x, axis=-1, keepdims=True)
    v = jnp.mean((x - m) ** 2, axis=-1, keepdims=True)
    return (x - m) * lax.rsqrt(v + eps) * g + b


def _full(shape):
    return pl.BlockSpec(shape, lambda i: tuple(0 for _ in shape))


def _node_body(x_ref, dummy, g0, b0, Adt, Ast, Wst, bs, Wgt, bg,
               pd_ref, ps_ref, gate_ref, gskip_ref):
    x = x_ref[...]
    bad = x[:, 0:1] == -999.0
    x = jnp.where(bad, dummy[...], x)
    xn = _ln_k(x, g0[...], b0[...])
    pd_ref[...] = xn @ Adt[...]
    ps_ref[...] = xn @ Ast[...]
    skip = xn @ Wst[...] + bs[...]
    gate = jax.nn.sigmoid(skip @ Wgt[...] + bg[...])
    gate_ref[...] = gate
    gskip_ref[...] = gate * skip


def _node(x, *ws):
    specs = [pl.BlockSpec((BN, IND), lambda i: (i, 0))]
    specs += [_full(w.shape) for w in ws]
    return pl.pallas_call(
        _node_body,
        grid=(GN,),
        in_specs=specs,
        out_specs=[
            pl.BlockSpec((BN, H1), lambda i: (i, 0)),
            pl.BlockSpec((BN, H1), lambda i: (i, 0)),
            pl.BlockSpec((BN, OUTD), lambda i: (i, 0)),
            pl.BlockSpec((BN, OUTD), lambda i: (i, 0)),
        ],
        out_shape=[
            jax.ShapeDtypeStruct((N, H1), f32),
            jax.ShapeDtypeStruct((N, H1), f32),
            jax.ShapeDtypeStruct((N, OUTD), f32),
            jax.ShapeDtypeStruct((N, OUTD), f32),
        ],
    )(x, *ws)


def _mlp1_body(pre, ea, ge, be, W1t, b1, W2t, b2, W3t, b3, Wc1t, bc1,
               Wc2t, bc2, Aet, b1a, W1bt, b1b, W1ct, b1c, out, ee_out):
    a = ea[...]
    h = _ln_k(a, ge[...], be[...])
    h = jnp.maximum(h @ W1t[...] + b1[...], 0.0)
    h = jnp.maximum(h @ W2t[...] + b2[...], 0.0)
    enc = h @ W3t[...] + b3[...]
    cw = jnp.maximum(a @ Wc1t[...] + bc1[...], 0.0)
    w = jax.nn.sigmoid(cw @ Wc2t[...] + bc2[...])
    e = enc * w
    ee_out[...] = e
    m = jnp.maximum(pre[...][:, :H1] + e @ Aet[...] + b1a[...], 0.0)
    m = jnp.maximum(m @ W1bt[...] + b1b[...], 0.0)
    m = m @ W1ct[...] + b1c[...]
    colid = lax.broadcasted_iota(i32, (BE, IND - H1), 1)
    aug = jnp.where(colid == 0, 1.0, 0.0).astype(f32)
    out[...] = jnp.concatenate([m, aug], axis=1)


def _mlp1(pre, ea, *ws):
    specs = [
        pl.BlockSpec((BE, IND), lambda i: (i, 0)),
        pl.BlockSpec((BE, EDGED), lambda i: (i, 0)),
    ]
    specs += [_full(w.shape) for w in ws]
    return pl.pallas_call(
        _mlp1_body,
        grid=(GE,),
        in_specs=specs,
        out_specs=[
            pl.BlockSpec((BE, IND), lambda i: (i, 0)),
            pl.BlockSpec((BE, OUTD), lambda i: (i, 0)),
        ],
        out_shape=[
            jax.ShapeDtypeStruct((E, IND), f32),
            jax.ShapeDtypeStruct((E, OUTD), f32),
        ],
    )(pre, ea, *ws)


def _x1_body(pa, pb, g1, b1, Bdt, Bst, qd_ref, qs_ref, invd_ref):
    s = pa[0] + pb[0]
    cnt = s[:, H1:H1 + 1]
    invd = 1.0 / jnp.maximum(cnt, 1.0)
    z = _ln_k(s[:, :H1] * invd, g1[...], b1[...])
    z = jnp.where(z >= 0.0, z, 0.01 * z)
    qd_ref[...] = z @ Bdt[...]
    qs_ref[...] = z @ Bst[...]
    invd_ref[...] = invd


def _x1(s1, g1, b1, Bdt, Bst):
    return pl.pallas_call(
        _x1_body,
        grid=(GN,),
        in_specs=[
            pl.BlockSpec((1, BN, IND), lambda i: (0, i, 0)),
            pl.BlockSpec((1, BN, IND), lambda i: (1, i, 0)),
            _full(g1.shape),
            _full(b1.shape),
            _full(Bdt.shape),
            _full(Bst.shape),
        ],
        out_specs=[
            pl.BlockSpec((BN, OUTD), lambda i: (i, 0)),
            pl.BlockSpec((BN, OUTD), lambda i: (i, 0)),
            pl.BlockSpec((BN, 1), lambda i: (i, 0)),
        ],
        out_shape=[
            jax.ShapeDtypeStruct((N, OUTD), f32),
            jax.ShapeDtypeStruct((N, OUTD), f32),
            jax.ShapeDtypeStruct((N, 1), f32),
        ],
    )(s1, s1, g1, b1, Bdt, Bst)


def _mlp2_body(pre, ee, Bet, b2a, W2bt, b2b, W2ct, b2c, out):
    e = ee[...]
    m = jnp.maximum(pre[...][:, :OUTD] + e @ Bet[...] + b2a[...], 0.0)
    m = jnp.maximum(m @ W2bt[...] + b2b[...], 0.0)
    m = m @ W2ct[...] + b2c[...]
    out[...] = jnp.concatenate([m, e], axis=1)


def _mlp2(pre, ee, *ws):
    specs = [
        pl.BlockSpec((BE, IND), lambda i: (i, 0)),
        pl.BlockSpec((BE, OUTD), lambda i: (i, 0)),
    ]
    specs += [_full(w.shape) for w in ws]
    return pl.pallas_call(
        _mlp2_body,
        grid=(GE,),
        in_specs=specs,
        out_specs=pl.BlockSpec((BE, IND), lambda i: (i, 0)),
        out_shape=jax.ShapeDtypeStruct((E, IND), f32),
    )(pre, ee, *ws)


def _final_body(p2a, p2b, invd, gate, gskip, g2, b2,
                Wp1t, bp1, Wp2t, bp2, Wp3t, bp3, xfc_ref, probs_ref):
    inv = invd[...]
    s = (p2a[0] + p2b[0]) * inv
    x2 = _ln_k(s[:, :OUTD], g2[...], b2[...])
    x2 = jnp.maximum(x2, 0.0)
    efm = s[:, OUTD:]
    g = gate[...]
    xf = gskip[...] + (1.0 - g) * x2
    xfc = jnp.concatenate([xf, efm], axis=1)
    xfc_ref[...] = xfc
    h = xfc @ Wp1t[...] + bp1[...]
    h = jnp.where(h > 0.0, h, jnp.exp(h) - 1.0)
    h = h @ Wp2t[...] + bp2[...]
    h = jnp.where(h > 0.0, h, jnp.exp(h) - 1.0)
    probs_ref[...] = h @ Wp3t[...] + bp3[...]


def _final(s2, invd, gate, gskip, *ws):
    specs = [
        pl.BlockSpec((1, BN, IND), lambda i: (0, i, 0)),
        pl.BlockSpec((1, BN, IND), lambda i: (1, i, 0)),
        pl.BlockSpec((BN, 1), lambda i: (i, 0)),
        pl.BlockSpec((BN, OUTD), lambda i: (i, 0)),
        pl.BlockSpec((BN, OUTD), lambda i: (i, 0)),
    ]
    specs += [_full(w.shape) for w in ws]
    return pl.pallas_call(
        _final_body,
        grid=(GN,),
        in_specs=specs,
        out_specs=[
            pl.BlockSpec((BN, 2 * OUTD), lambda i: (i, 0)),
            pl.BlockSpec((BN, 1), lambda i: (i, 0)),
        ],
        out_shape=[
            jax.ShapeDtypeStruct((N, 2 * OUTD), f32),
            jax.ShapeDtypeStruct((N, 1), f32),
        ],
    )(s2, s2, invd, gate, gskip, *ws)


_sc_mesh = plsc.VectorSubcoreMesh(
    core_axis_name="c", subcore_axis_name="s", num_cores=NC, num_subcores=NS)


NB = 5
GRP = NCHUNK // NB
CHG = 200
GRPG = EPW // (CHG * NB)


def _make_gather_add(W):
    @functools.partial(
        pl.kernel,
        out_type=jax.ShapeDtypeStruct((E, IND), f32),
        mesh=_sc_mesh,
        scratch_types=([pltpu.VMEM((CHG,), i32)] * (2 * NB)
                       + [pltpu.VMEM((CHG, W), f32)] * NB
                       + [pltpu.SemaphoreType.DMA] * 3),
        compiler_params=pltpu.CompilerParams(use_tc_tiling_on_sc=False),
    )
    def gather_add(ta, tb, ia, ib, out, *scr):
        idxa = scr[:NB]
        idxb = scr[NB:2 * NB]
        rows = scr[2 * NB:3 * NB]
        semi, semg, semw = scr[3 * NB:]
        c = lax.axis_index("c")
        s = lax.axis_index("s")
        wid = c * NS + s

        def group(g, carry):
            bases = [wid * EPW + (g * NB + b) * CHG for b in range(NB)]
            cps = []
            for b in range(NB):
                cps.append(pltpu.async_copy(ia.at[pl.ds(bases[b], CHG)],
                                            idxa[b], semi))
                cps.append(pltpu.async_copy(ib.at[pl.ds(bases[b], CHG)],
                                            idxb[b], semi))
            for cp in cps:
                cp.wait()
            cps = [pltpu.async_copy(ta.at[idxa[b]], rows[b], semg)
                   for b in range(NB)]
            for cp in cps:
                cp.wait()
            cps = [pltpu.async_copy(tb.at[idxb[b]], rows[b], semg, add=True)
                   for b in range(NB)]
            for cp in cps:
                cp.wait()
            cps = [pltpu.async_copy(
                rows[b], out.at[pl.ds(bases[b], CHG), pl.ds(0, W)], semw)
                for b in range(NB)]
            for cp in cps:
                cp.wait()
            return carry

        lax.fori_loop(0, GRPG, group, 0)

    return gather_add


_sc_gather_add32 = _make_gather_add(H1)
_sc_gather_add64 = _make_gather_add(OUTD)


NBS = 4
NGPAIR = 31


@functools.partial(
    pl.kernel,
    out_type=jax.ShapeDtypeStruct((NC, NPAD, IND), f32),
    mesh=_sc_mesh,
    scratch_types=([pltpu.VMEM((CH,), i32)] * NBS
                   + [pltpu.VMEM((CH, IND), f32)] * NBS
                   + [pltpu.VMEM_SHARED((NPAD, IND), f32)]
                   + [pltpu.SemaphoreType.DMA] * 4),
)
def _sc_scatter128(vals, dsti, zer, out, *scr):
    idx = scr[:NBS]
    rows = scr[NBS:2 * NBS]
    acc = scr[2 * NBS]
    semia, semib, semsa, semsb = scr[2 * NBS + 1:]
    c = lax.axis_index("c")
    s = lax.axis_index("s")
    r0 = s * RPT
    pltpu.sync_copy(zer.at[pl.ds(r0, RPT)], acc.at[pl.ds(r0, RPT)])
    plsc.subcore_barrier()
    wid = c * NS + s
    base0 = wid * EPW

    def fire_loads(g, sl, sem):
        cps = []
        for k in range(2):
            b = g * 2 + k
            cps.append(pltpu.async_copy(
                dsti.at[pl.ds(base0 + b * CH, CH)], idx[sl + k], sem))
            cps.append(pltpu.async_copy(
                vals.at[pl.ds(base0 + b * CH, CH)], rows[sl + k], sem))
        return cps

    def drain_loads(sl, sem):
        for k in range(2):
            pltpu.make_async_copy(dsti.at[pl.ds(base0, CH)],
                                  idx[sl + k], sem).wait()
            pltpu.make_async_copy(vals.at[pl.ds(base0, CH)],
                                  rows[sl + k], sem).wait()

    def fire_scats(sl, sem):
        return [pltpu.async_copy(rows[sl + k], acc.at[idx[sl + k]], sem,
                                 add=True) for k in range(2)]

    fire_loads(0, 0, semia)

    def pair(gg, carry):
        g0 = 2 * gg
        cps_b = fire_loads(g0 + 1, 2, semib)
        drain_loads(0, semia)
        sa = fire_scats(0, semsa)
        for cp in sa:
            cp.wait()

        @pl.when(gg + 1 < NGPAIR)
        def _():
            fire_loads(g0 + 2, 0, semia)

        for cp in cps_b:
            cp.wait()
        sb = fire_scats(2, semsb)
        for cp in sb:
            cp.wait()
        return carry

    lax.fori_loop(0, NGPAIR, pair, 0)

    tail = base0 + NGPAIR * 2 * 2 * CH
    pltpu.sync_copy(dsti.at[pl.ds(tail, CH)], idx[0])
    pltpu.sync_copy(vals.at[pl.ds(tail, CH)], rows[0])
    pltpu.sync_copy(rows[0], acc.at[idx[0]], add=True)
    plsc.subcore_barrier()
    pltpu.sync_copy(acc.at[pl.ds(r0, RPT)], out.at[c, pl.ds(r0, RPT)])


def kernel(x_in, edge_index, edge_attr, params):
    p = params
    src = edge_index[0, 0]
    dst = edge_index[0, 1]
    x = x_in[0]
    ea = edge_attr[0]

    def r(v):
        return v.reshape(1, -1)

    Adt = p['Wm1a'][:, :IND].T
    Ast = p['Wm1a'][:, IND:2 * IND].T
    Aet = p['Wm1a'][:, 2 * IND:].T
    Bdt = p['Wm2a'][:, :H1].T
    Bst = p['Wm2a'][:, H1:2 * H1].T
    Bet = p['Wm2a'][:, 2 * H1:].T
    zer = jnp.zeros((NPAD, IND), f32)

    pd, ps, gate, gskip = _node(x, r(p['dummy']), r(p['g0']), r(p['b0']),
                                Adt, Ast,
                                p['Wskip'].T, r(p['bskip']),
                                p['Wg'].T, r(p['bg']))
    m1pre = _sc_gather_add32(pd, ps, dst, src)
    m1, e_enc = _mlp1(m1pre, ea,
                      r(p['ge']), r(p['be']),
                      p['We1'].T, r(p['be1']), p['We2'].T, r(p['be2']),
                      p['We3'].T, r(p['be3']),
                      p['Wc1'].T, r(p['bc1']), p['Wc2'].T, r(p['bc2']),
                      Aet, r(p['bm1a']),
                      p['Wm1b'].T, r(p['bm1b']), p['Wm1c'].T, r(p['bm1c']))
    s1 = _sc_scatter128(m1, dst, zer)
    qd, qs, invd = _x1(s1, r(p['g1']), r(p['b1']), Bdt, Bst)
    m2pre = _sc_gather_add64(qd, qs, dst, src)
    m2 = _mlp2(m2pre, e_enc, Bet, r(p['bm2a']),
               p['Wm2b'].T, r(p['bm2b']), p['Wm2c'].T, r(p['bm2c']))
    s2 = _sc_scatter128(m2, dst, zer)
    xfc, probs = _final(s2, invd, gate, gskip,
                        r(p['g2']), r(p['b2']),
                        p['Wp1'].T, r(p['bp1']), p['Wp2'].T, r(p['bp2']),
                        p['Wp3'].T, r(p['bp3']))
    return (xfc[None], probs[None], jnp.zeros((1,), f32))
